# baseline (speedup 1.0000x reference)
import functools

import jax
import jax.numpy as jnp
from jax.experimental import pallas as pl
from jax.experimental.pallas import tpu as pltpu

ENC_DIMS = [768, 2048, 1024, 512, 256]
NUM_EMB = 1024
E_DIM = 256
BETA = 0.25
B = 16384

BM_ENC = 256
BM_DEC = 512


def _dot(a, b, precision=None):
    return jax.lax.dot_general(a, b, (((1,), (0,)), ((), ())),
                               precision=precision,
                               preferred_element_type=jnp.float32)


def _dot_bf(a, b):
    return jax.lax.dot_general(a.astype(jnp.bfloat16), b.astype(jnp.bfloat16),
                               (((1,), (0,)), ((), ())),
                               preferred_element_type=jnp.float32)


def _dot_t(a, b, precision=None):
    return jax.lax.dot_general(a, b, (((1,), (1,)), ((), ())),
                               precision=precision,
                               preferred_element_type=jnp.float32)


def _enc_vq_kernel(x_ref, ew0, eb0, ew1, eb1, ew2, eb2, ew3, eb3,
                   cb0, cb1, cb2, cb3,
                   xq_ref, idx_ref, loss_ref,
                   acc_ref, cbsq_ref):
    i = pl.program_id(0)
    nsteps = pl.num_programs(0)
    cbs = [cb0, cb1, cb2, cb3]

    @pl.when(i == 0)
    def _init():
        for l in range(4):
            c = cbs[l][...]
            sq = jnp.sum(c * c, axis=1)
            cbsq_ref[l, :] = sq
        acc_ref[...] = jnp.zeros_like(acc_ref)

    h = x_ref[...]
    h = jnp.maximum(_dot(h, ew0[...]) + eb0[...], 0.0)
    h = jnp.maximum(_dot(h, ew1[...]) + eb1[...], 0.0)
    h = jnp.maximum(_dot(h, ew2[...]) + eb2[...], 0.0)
    z = _dot(h, ew3[...]) + eb3[...]

    r = z
    xq = jnp.zeros_like(z)
    acc = jnp.zeros((1, 128), dtype=jnp.float32)
    iota = jax.lax.broadcasted_iota(jnp.int32, (BM_ENC, NUM_EMB), 1)
    for l in range(4):
        cb = cbs[l][...]
        s = _dot_t(r, cb)
        latsq = jnp.sum(r * r, axis=1, keepdims=True)
        d = (latsq + cbsq_ref[l, :][None, :]) - 2.0 * s
        idx = jnp.argmin(d, axis=1)
        onehot = (iota == idx[:, None]).astype(jnp.float32)
        x_res = _dot(onehot, cb, precision=jax.lax.Precision.HIGHEST)
        r = r - x_res
        xq = xq + x_res
        r2 = jnp.sum(r * r, axis=0)
        acc = acc + (r2[:128] + r2[128:])[None, :]
        idx_ref[l, :] = idx

    acc_ref[...] += acc
    xq_ref[...] = xq

    @pl.when(i == nsteps - 1)
    def _fin():
        coef = (1.0 + BETA) / (4.0 * B * E_DIM)
        loss_ref[...] = jnp.sum(acc_ref[...] * coef, axis=1, keepdims=True)


def _dec_kernel(xq_ref, dw0, db0, dw1, db1, dw2, db2, dw3, db3, out_ref):
    h = xq_ref[...]
    h = jnp.maximum(_dot(h, dw0[...]) + db0[...], 0.0)
    h = jnp.maximum(_dot(h, dw1[...]) + db1[...], 0.0)
    h = jnp.maximum(_dot(h, dw2[...]) + db2[...], 0.0)
    out_ref[...] = _dot(h, dw3[...]) + db3[...]


def _full(shape):
    return pl.BlockSpec(shape, lambda i: tuple(0 for _ in shape))


def kernel(x, ew0, eb0, ew1, eb1, ew2, eb2, ew3, eb3,
           dw0, db0, dw1, db1, dw2, db2, dw3, db3,
           cb0, cb1, cb2, cb3):
    enc_ws = [ew0, ew1, ew2, ew3]
    enc_bs = [eb0.reshape(1, -1), eb1.reshape(1, -1),
              eb2.reshape(1, -1), eb3.reshape(1, -1)]
    dec_ws = [dw0, dw1, dw2, dw3]
    dec_bs = [db0.reshape(1, -1), db1.reshape(1, -1),
              db2.reshape(1, -1), db3.reshape(1, -1)]
    cbs = [cb0, cb1, cb2, cb3]

    n_enc = B // BM_ENC
    enc_in_specs = [pl.BlockSpec((BM_ENC, ENC_DIMS[0]), lambda i: (i, 0))]
    for w, b in zip(enc_ws, enc_bs):
        enc_in_specs.append(_full(w.shape))
        enc_in_specs.append(_full(b.shape))
    enc_in_specs += [_full(c.shape) for c in cbs]

    xq, idx4, loss = pl.pallas_call(
        _enc_vq_kernel,
        grid=(n_enc,),
        in_specs=enc_in_specs,
        out_specs=[
            pl.BlockSpec((BM_ENC, E_DIM), lambda i: (i, 0)),
            pl.BlockSpec((4, BM_ENC), lambda i: (0, i)),
            pl.BlockSpec((1, 1), lambda i: (0, 0)),
        ],
        out_shape=[
            jax.ShapeDtypeStruct((B, E_DIM), jnp.float32),
            jax.ShapeDtypeStruct((4, B), jnp.int32),
            jax.ShapeDtypeStruct((1, 1), jnp.float32),
        ],
        scratch_shapes=[
            pltpu.VMEM((1, 128), jnp.float32),
            pltpu.VMEM((4, NUM_EMB), jnp.float32),
        ],
    )(x, ew0, enc_bs[0], ew1, enc_bs[1], ew2, enc_bs[2], ew3, enc_bs[3],
      *cbs)

    n_dec = B // BM_DEC
    dec_in_specs = [pl.BlockSpec((BM_DEC, E_DIM), lambda i: (i, 0))]
    for w, b in zip(dec_ws, dec_bs):
        dec_in_specs.append(_full(w.shape))
        dec_in_specs.append(_full(b.shape))

    out = pl.pallas_call(
        _dec_kernel,
        grid=(n_dec,),
        in_specs=dec_in_specs,
        out_specs=pl.BlockSpec((BM_DEC, ENC_DIMS[0]), lambda i: (i, 0)),
        out_shape=jax.ShapeDtypeStruct((B, ENC_DIMS[0]), jnp.float32),
    )(xq, dw0, dec_bs[0], dw1, dec_bs[1], dw2, dec_bs[2], dw3, dec_bs[3])

    rq_loss = loss[0, 0]
    all_indices = idx4.T
    return (out, rq_loss, all_indices, xq)


# speedup vs baseline: 1.3834x; 1.3834x over previous
import functools

import jax
import jax.numpy as jnp
from jax.experimental import pallas as pl
from jax.experimental.pallas import tpu as pltpu

ENC_DIMS = [768, 2048, 1024, 512, 256]
NUM_EMB = 1024
E_DIM = 256
BETA = 0.25
B = 16384

BM_ENC = 256
BM_DEC = 512


def _dot(a, b, precision=None):
    return jax.lax.dot_general(a, b, (((1,), (0,)), ((), ())),
                               precision=precision,
                               preferred_element_type=jnp.float32)


def _dot_bf(a, b):
    return jax.lax.dot_general(a.astype(jnp.bfloat16), b.astype(jnp.bfloat16),
                               (((1,), (0,)), ((), ())),
                               preferred_element_type=jnp.float32)


def _dot_t(a, b, precision=None):
    return jax.lax.dot_general(a, b, (((1,), (1,)), ((), ())),
                               precision=precision,
                               preferred_element_type=jnp.float32)


def _enc_vq_kernel(x_ref, ew0, eb0, ew1, eb1, ew2, eb2, ew3, eb3,
                   cb0, cb1, cb2, cb3,
                   xq_ref, idx_ref, loss_ref,
                   acc_ref, cbsq_ref, cbhi_ref, cblo_ref):
    i = pl.program_id(0)
    nsteps = pl.num_programs(0)
    cbs = [cb0, cb1, cb2, cb3]

    @pl.when(i == 0)
    def _init():
        for l in range(4):
            c = cbs[l][...]
            sq = jnp.sum(c * c, axis=1)
            cbsq_ref[l, :] = sq
            hi = c.astype(jnp.bfloat16)
            cbhi_ref[l, :, :] = hi
            cblo_ref[l, :, :] = (c - hi.astype(jnp.float32)).astype(jnp.bfloat16)
        acc_ref[...] = jnp.zeros_like(acc_ref)

    h = x_ref[...]
    h = jnp.maximum(_dot(h, ew0[...]) + eb0[...], 0.0)
    h = jnp.maximum(_dot(h, ew1[...]) + eb1[...], 0.0)
    h = jnp.maximum(_dot(h, ew2[...]) + eb2[...], 0.0)
    z = _dot(h, ew3[...]) + eb3[...]

    r = z
    xq = jnp.zeros_like(z)
    acc = jnp.zeros((1, 128), dtype=jnp.float32)
    iota = jax.lax.broadcasted_iota(jnp.int32, (BM_ENC, NUM_EMB), 1)
    for l in range(4):
        cb = cbs[l][...]
        s = _dot_t(r, cb)
        latsq = jnp.sum(r * r, axis=1, keepdims=True)
        d = (latsq + cbsq_ref[l, :][None, :]) - 2.0 * s
        idx = jnp.argmin(d, axis=1)
        onehot = (iota == idx[:, None]).astype(jnp.bfloat16)
        x_res = (_dot(onehot, cbhi_ref[l, :, :])
                 + _dot(onehot, cblo_ref[l, :, :]))
        r = r - x_res
        xq = xq + x_res
        r2 = jnp.sum(r * r, axis=0)
        acc = acc + (r2[:128] + r2[128:])[None, :]
        idx_ref[l, :] = idx

    acc_ref[...] += acc
    xq_ref[...] = xq

    @pl.when(i == nsteps - 1)
    def _fin():
        coef = (1.0 + BETA) / (4.0 * B * E_DIM)
        loss_ref[...] = jnp.sum(acc_ref[...] * coef, axis=1, keepdims=True)


def _dec_kernel(xq_ref, dw0, db0, dw1, db1, dw2, db2, dw3, db3, out_ref):
    h = xq_ref[...]
    h = jnp.maximum(_dot_bf(h, dw0[...]) + db0[...], 0.0)
    h = jnp.maximum(_dot_bf(h, dw1[...]) + db1[...], 0.0)
    h = jnp.maximum(_dot_bf(h, dw2[...]) + db2[...], 0.0)
    out_ref[...] = _dot_bf(h, dw3[...]) + db3[...]


def _full(shape):
    return pl.BlockSpec(shape, lambda i: tuple(0 for _ in shape))


def kernel(x, ew0, eb0, ew1, eb1, ew2, eb2, ew3, eb3,
           dw0, db0, dw1, db1, dw2, db2, dw3, db3,
           cb0, cb1, cb2, cb3):
    enc_ws = [ew0, ew1, ew2, ew3]
    enc_bs = [eb0.reshape(1, -1), eb1.reshape(1, -1),
              eb2.reshape(1, -1), eb3.reshape(1, -1)]
    dec_ws = [dw0, dw1, dw2, dw3]
    dec_bs = [db0.reshape(1, -1), db1.reshape(1, -1),
              db2.reshape(1, -1), db3.reshape(1, -1)]
    cbs = [cb0, cb1, cb2, cb3]

    n_enc = B // BM_ENC
    enc_in_specs = [pl.BlockSpec((BM_ENC, ENC_DIMS[0]), lambda i: (i, 0))]
    for w, b in zip(enc_ws, enc_bs):
        enc_in_specs.append(_full(w.shape))
        enc_in_specs.append(_full(b.shape))
    enc_in_specs += [_full(c.shape) for c in cbs]

    xq, idx4, loss = pl.pallas_call(
        _enc_vq_kernel,
        grid=(n_enc,),
        in_specs=enc_in_specs,
        out_specs=[
            pl.BlockSpec((BM_ENC, E_DIM), lambda i: (i, 0)),
            pl.BlockSpec((4, BM_ENC), lambda i: (0, i)),
            pl.BlockSpec((1, 1), lambda i: (0, 0)),
        ],
        out_shape=[
            jax.ShapeDtypeStruct((B, E_DIM), jnp.float32),
            jax.ShapeDtypeStruct((4, B), jnp.int32),
            jax.ShapeDtypeStruct((1, 1), jnp.float32),
        ],
        scratch_shapes=[
            pltpu.VMEM((1, 128), jnp.float32),
            pltpu.VMEM((4, NUM_EMB), jnp.float32),
            pltpu.VMEM((4, NUM_EMB, E_DIM), jnp.bfloat16),
            pltpu.VMEM((4, NUM_EMB, E_DIM), jnp.bfloat16),
        ],
    )(x, ew0, enc_bs[0], ew1, enc_bs[1], ew2, enc_bs[2], ew3, enc_bs[3],
      *cbs)

    n_dec = B // BM_DEC
    dec_in_specs = [pl.BlockSpec((BM_DEC, E_DIM), lambda i: (i, 0))]
    for w, b in zip(dec_ws, dec_bs):
        dec_in_specs.append(_full(w.shape))
        dec_in_specs.append(_full(b.shape))

    out = pl.pallas_call(
        _dec_kernel,
        grid=(n_dec,),
        in_specs=dec_in_specs,
        out_specs=pl.BlockSpec((BM_DEC, ENC_DIMS[0]), lambda i: (i, 0)),
        out_shape=jax.ShapeDtypeStruct((B, ENC_DIMS[0]), jnp.float32),
    )(xq, dw0, dec_bs[0], dw1, dec_bs[1], dw2, dec_bs[2], dw3, dec_bs[3])

    rq_loss = loss[0, 0]
    all_indices = idx4.T
    return (out, rq_loss, all_indices, xq)
